# baseline (speedup 1.0000x reference)
import jax
import jax.numpy as jnp
from jax.experimental import pallas as pl
from jax.experimental.pallas import tpu as pltpu

NUM_CLASSES = 1000
B, S = 1024, 20
N = B * S
CP = 1024
CHUNK = 1280
NCHUNK = N // CHUNK
NBUF = 8


def _onehot_stream(x_ref, o_hbm, *scratch):
    bufs = scratch[:NBUF]
    sems = scratch[NBUF:]
    copies = [None] * NBUF
    for c in range(NCHUNK):
        k = c % NBUF
        if copies[k] is not None:
            copies[k].wait()
        idx = x_ref[pl.ds(c * CHUNK, CHUNK)]
        cols = jax.lax.broadcasted_iota(jnp.int32, (CHUNK, CP), 1)
        bufs[k][...] = (cols == idx[:, None]).astype(jnp.int8)
        cp = pltpu.make_async_copy(bufs[k], o_hbm.at[pl.ds(c * CHUNK, CHUNK)], sems[k])
        cp.start()
        copies[k] = cp
    for k in range(NBUF):
        copies[k].wait()


def kernel(x, table):
    del table
    onehot8 = pl.pallas_call(
        _onehot_stream,
        in_specs=[pl.BlockSpec(memory_space=pltpu.VMEM)],
        out_specs=pl.BlockSpec(memory_space=pltpu.HBM),
        out_shape=jax.ShapeDtypeStruct((N, CP), jnp.int8),
        scratch_shapes=(
            [pltpu.VMEM((CHUNK, CP), jnp.int8) for _ in range(NBUF)]
            + [pltpu.SemaphoreType.DMA for _ in range(NBUF)]
        ),
    )(x.reshape(N))
    return onehot8[:, :NUM_CLASSES].astype(jnp.float32).reshape(B, S, NUM_CLASSES)


# speedup vs baseline: 1.5835x; 1.5835x over previous
import jax
import jax.numpy as jnp
from jax.experimental import pallas as pl
from jax.experimental.pallas import tpu as pltpu

NUM_CLASSES = 1000
B, S = 1024, 20
SP = 32
CP = 1024
CB = 64
NCHUNK = B // CB
NBUF = 8


def _onehot_stream(x_ref, o_hbm, idx_pad, *scratch):
    bufs = scratch[:NBUF]
    sems = scratch[NBUF:]
    idx_pad[:, 0:S] = x_ref[...]
    idx_pad[:, S:SP] = jnp.full((B, SP - S), -1, jnp.int32)
    copies = [None] * NBUF
    for c in range(NCHUNK):
        k = c % NBUF
        if copies[k] is not None:
            copies[k].wait()
        idx = idx_pad[pl.ds(c * CB, CB), :]
        cols = jax.lax.broadcasted_iota(jnp.int32, (CB, SP, CP), 2)
        bufs[k][...] = (cols == idx[:, :, None]).astype(jnp.int8)
        cp = pltpu.make_async_copy(bufs[k], o_hbm.at[pl.ds(c * CB, CB)], sems[k])
        cp.start()
        copies[k] = cp
    for k in range(NBUF):
        copies[k].wait()


def kernel(x, table):
    del table
    onehot8 = pl.pallas_call(
        _onehot_stream,
        in_specs=[pl.BlockSpec(memory_space=pltpu.VMEM)],
        out_specs=pl.BlockSpec(memory_space=pltpu.HBM),
        out_shape=jax.ShapeDtypeStruct((B, SP, CP), jnp.int8),
        scratch_shapes=(
            [pltpu.VMEM((B, SP), jnp.int32)]
            + [pltpu.VMEM((CB, SP, CP), jnp.int8) for _ in range(NBUF)]
            + [pltpu.SemaphoreType.DMA for _ in range(NBUF)]
        ),
    )(x)
    return onehot8[:, :S, :NUM_CLASSES].astype(jnp.float32)


# speedup vs baseline: 1.8484x; 1.1673x over previous
import jax
import jax.numpy as jnp
from jax.experimental import pallas as pl
from jax.experimental.pallas import tpu as pltpu

NUM_CLASSES = 1000
B, S = 1024, 20
SA = 16
CA = 896
CB = 64
NCHUNK = B // CB
NBUF = 4


def _onehot_stream(x_ref, o_hbm, *scratch):
    bufs = scratch[: 3 * NBUF]
    sems = scratch[3 * NBUF :]
    copies = [None] * (3 * NBUF)
    for c in range(NCHUNK):
        k = c % NBUF
        for j in range(3):
            if copies[3 * k + j] is not None:
                copies[3 * k + j].wait()
        idx = x_ref[pl.ds(c * CB, CB), :]
        ba, bb, bc = bufs[3 * k], bufs[3 * k + 1], bufs[3 * k + 2]
        ia = idx[:, :SA, None]
        ba[...] = (
            jax.lax.broadcasted_iota(jnp.int32, (CB, SA, CA), 2) == ia
        ).astype(jnp.float32)
        bb[...] = (
            jax.lax.broadcasted_iota(jnp.int32, (CB, SA, NUM_CLASSES - CA), 2) + CA
            == ia
        ).astype(jnp.float32)
        bc[...] = (
            jax.lax.broadcasted_iota(jnp.int32, (CB, S - SA, NUM_CLASSES), 2)
            == idx[:, SA:S, None]
        ).astype(jnp.float32)
        row = pl.ds(c * CB, CB)
        dsts = (
            o_hbm.at[row, pl.ds(0, SA), pl.ds(0, CA)],
            o_hbm.at[row, pl.ds(0, SA), pl.ds(CA, NUM_CLASSES - CA)],
            o_hbm.at[row, pl.ds(SA, S - SA), pl.ds(0, NUM_CLASSES)],
        )
        for j, (src, dst) in enumerate(zip((ba, bb, bc), dsts)):
            cp = pltpu.make_async_copy(src, dst, sems[3 * k + j])
            cp.start()
            copies[3 * k + j] = cp
    for cp in copies:
        cp.wait()


def kernel(x, table):
    del table
    return pl.pallas_call(
        _onehot_stream,
        in_specs=[pl.BlockSpec(memory_space=pltpu.VMEM)],
        out_specs=pl.BlockSpec(memory_space=pltpu.HBM),
        out_shape=jax.ShapeDtypeStruct((B, S, NUM_CLASSES), jnp.float32),
        scratch_shapes=(
            [
                buf
                for _ in range(NBUF)
                for buf in (
                    pltpu.VMEM((CB, SA, CA), jnp.float32),
                    pltpu.VMEM((CB, SA, NUM_CLASSES - CA), jnp.float32),
                    pltpu.VMEM((CB, S - SA, NUM_CLASSES), jnp.float32),
                )
            ]
            + [pltpu.SemaphoreType.DMA for _ in range(3 * NBUF)]
        ),
    )(x)
